# Initial kernel scaffold; baseline (speedup 1.0000x reference)
import jax
import jax.numpy as jnp
from jax.experimental import pallas as pl


def kernel(p, x, o, n, idx, Wq, Wk, Wv, W1, b1, W2, b2, Wo, bo):
    raise NotImplementedError("write your pallas kernel here")


# speedup vs baseline: 1.1608x; 1.1608x over previous
import functools

import jax
import jax.numpy as jnp
from jax import lax
from jax.experimental import pallas as pl
from jax.experimental.pallas import tpu as pltpu
from jax.experimental.pallas import tpu_sc as plsc

N, C_IN, C_HID, C_OUT, K, H = 16384, 256, 256, 256, 16, 4
DH = C_HID // H
NK = N * K

NC, NS = 2, 16
NW = NC * NS


def _qkv_body(x_ref, w_ref, q_ref, kv_ref):
    y = jnp.dot(x_ref[...], w_ref[...], preferred_element_type=jnp.float32)
    q_ref[...] = y[:, :C_HID]
    kv_ref[...] = y[:, C_HID:]


def _qkv_call(x, wqkv):
    rb = 2048
    return pl.pallas_call(
        _qkv_body,
        grid=(N // rb,),
        in_specs=[
            pl.BlockSpec((rb, C_IN), lambda i: (i, 0)),
            pl.BlockSpec((C_IN, 3 * C_HID), lambda i: (0, 0)),
        ],
        out_specs=[
            pl.BlockSpec((rb, C_HID), lambda i: (i, 0)),
            pl.BlockSpec((rb, 2 * C_HID), lambda i: (i, 0)),
        ],
        out_shape=[
            jax.ShapeDtypeStruct((N, C_HID), jnp.float32),
            jax.ShapeDtypeStruct((N, 2 * C_HID), jnp.float32),
        ],
    )(x, wqkv)


_GC = 128
_ROWS_W = NK // NW


def _gather_kv_body(kv_hbm, idx_hbm, kvr_hbm, idx_v, kv_v, sem_g):
    wid = lax.axis_index("s") * NC + lax.axis_index("c")
    base0 = wid * _ROWS_W

    def chunk(c, _):
        base = base0 + c * _GC
        pltpu.sync_copy(idx_hbm.at[pl.ds(base, _GC)], idx_v)
        pltpu.async_copy(kv_hbm.at[idx_v], kv_v, sem_g).wait()
        pltpu.sync_copy(kv_v, kvr_hbm.at[pl.ds(base, _GC)])
        return ()

    lax.fori_loop(0, _ROWS_W // _GC, chunk, (), unroll=False)


def _gather_kv_call(kv, idx_flat):
    mesh = plsc.VectorSubcoreMesh(core_axis_name="c", subcore_axis_name="s")
    f = pl.kernel(
        _gather_kv_body,
        out_type=jax.ShapeDtypeStruct((NK, 2 * C_HID), jnp.float32),
        mesh=mesh,
        scratch_types=[
            pltpu.VMEM((_GC,), jnp.int32),
            pltpu.VMEM((_GC, 2 * C_HID), jnp.float32),
            pltpu.SemaphoreType.DMA,
        ],
    )
    return f(kv, idx_flat)


_GEOM_C = 512


def _gather_geom_body(pn6_hbm, idx_hbm, pnr_hbm, tbl_v, idx_v, out_v):
    wid = lax.axis_index("s") * NC + lax.axis_index("c")
    base0 = wid * _ROWS_W
    pltpu.sync_copy(pn6_hbm, tbl_v)
    lanes = jnp.arange(16, dtype=jnp.int32)

    def outer(c, _):
        base = base0 + c * _GEOM_C
        pltpu.sync_copy(idx_hbm.at[pl.ds(base, _GEOM_C)], idx_v)

        def inner(j, _):
            iv = idx_v[pl.ds(j * 16, 16)]
            rows16 = (j * 16 + lanes) * 16
            for comp in range(6):
                g = plsc.load_gather(tbl_v, [comp * N + iv])
                plsc.store_scatter(out_v, [rows16 + comp], g)
            return ()

        lax.fori_loop(0, _GEOM_C // 16, inner, (), unroll=False)
        pltpu.sync_copy(out_v, pnr_hbm.at[pl.ds(base * 16, _GEOM_C * 16)])
        return ()

    lax.fori_loop(0, _ROWS_W // _GEOM_C, outer, (), unroll=False)


def _gather_geom_call(pn6_flat, idx_flat):
    mesh = plsc.VectorSubcoreMesh(core_axis_name="c", subcore_axis_name="s")
    f = pl.kernel(
        _gather_geom_body,
        out_type=jax.ShapeDtypeStruct((NK * 16,), jnp.float32),
        mesh=mesh,
        scratch_types=[
            pltpu.VMEM((6 * N,), jnp.float32),
            pltpu.VMEM((_GEOM_C,), jnp.int32),
            pltpu.VMEM((_GEOM_C * 16,), jnp.float32),
        ],
        compiler_params=pltpu.CompilerParams(needs_layout_passes=False),
    )
    return f(pn6_flat, idx_flat)


_P = 128


def _angle(ax, ay, az, bx, by, bz):
    cx = ay * bz - az * by
    cy = az * bx - ax * bz
    cz = ax * by - ay * bx
    cn = jnp.sqrt(cx * cx + cy * cy + cz * cz)
    dt = ax * bx + ay * by + az * bz
    return jnp.where((cn == 0.0) & (dt == 0.0), 0.0, jnp.arctan2(cn, dt))


def _main_body(q_ref, pnc_ref, pnr_ref, kv_ref, w1_ref, b1_ref, w2_ref,
               b2_ref, wo_ref, bo_ref, xout_ref, ppf_ref):
    pk = _P * K
    pnr = pnr_ref[...]
    pnc = pnc_ref[...]
    pncr = jnp.broadcast_to(pnc[:, None, :], (_P, K, 16)).reshape(pk, 16)

    pxr, pyr, pzr = pnr[:, 0:1], pnr[:, 1:2], pnr[:, 2:3]
    nxr, nyr, nzr = pnr[:, 3:4], pnr[:, 4:5], pnr[:, 5:6]
    pxc, pyc, pzc = pncr[:, 0:1], pncr[:, 1:2], pncr[:, 2:3]
    nxc, nyc, nzc = pncr[:, 3:4], pncr[:, 4:5], pncr[:, 5:6]

    dx, dy, dz = pxr - pxc, pyr - pyc, pzr - pzc
    nd = jnp.sqrt(dx * dx + dy * dy + dz * dz)
    inv = 1.0 / (nd + 1e-8)
    ux, uy, uz = dx * inv, dy * inv, dz * inv

    a1 = _angle(nxc, nyc, nzc, ux, uy, uz)
    a2 = _angle(nxr, nyr, nzr, ux, uy, uz)
    a3 = _angle(nxc, nyc, nzc, nxr, nyr, nzr)
    ppf = jnp.concatenate([a1, a2, a3, nd], axis=1)
    ppf_ref[...] = ppf

    h1 = jnp.maximum(
        jnp.dot(ppf, w1_ref[...], preferred_element_type=jnp.float32)
        + b1_ref[...], 0.0)
    pe = jnp.dot(h1.astype(jnp.bfloat16), w2_ref[...].astype(jnp.bfloat16),
                 preferred_element_type=jnp.float32) + b2_ref[...]

    kv = kv_ref[...]
    kr = kv[:, :C_HID] + pe
    vr = kv[:, C_HID:] + pe

    q = q_ref[...]
    qrep = jnp.broadcast_to(q[:, None, :], (_P, K, C_HID)).reshape(pk, C_HID)
    prod = qrep * kr
    logits = jnp.concatenate(
        [jnp.sum(prod[:, h * DH:(h + 1) * DH], axis=1, keepdims=True)
         for h in range(H)], axis=1) * (1.0 / 8.0)
    lg = logits.reshape(_P, K, H)
    m = jnp.max(lg, axis=1, keepdims=True)
    e = jnp.exp(lg - m)
    s = jnp.sum(e, axis=1, keepdims=True)
    attn = (e / s).reshape(pk, H)
    w256 = jnp.concatenate(
        [jnp.broadcast_to(attn[:, h:h + 1], (pk, DH)) for h in range(H)],
        axis=1)
    out = (w256 * vr).reshape(_P, K, C_HID).sum(axis=1)
    xout_ref[...] = jnp.dot(out, wo_ref[...],
                            preferred_element_type=jnp.float32) + bo_ref[...]


def _main_call(q, pn16, pnr, kvr, w1, b1, w2, b2, wo, bo):
    pk = _P * K
    return pl.pallas_call(
        _main_body,
        grid=(N // _P,),
        in_specs=[
            pl.BlockSpec((_P, C_HID), lambda i: (i, 0)),
            pl.BlockSpec((_P, 16), lambda i: (i, 0)),
            pl.BlockSpec((pk, 16), lambda i: (i, 0)),
            pl.BlockSpec((pk, 2 * C_HID), lambda i: (i, 0)),
            pl.BlockSpec((4, C_HID), lambda i: (0, 0)),
            pl.BlockSpec((1, C_HID), lambda i: (0, 0)),
            pl.BlockSpec((C_HID, C_HID), lambda i: (0, 0)),
            pl.BlockSpec((1, C_HID), lambda i: (0, 0)),
            pl.BlockSpec((C_HID, C_OUT), lambda i: (0, 0)),
            pl.BlockSpec((1, C_OUT), lambda i: (0, 0)),
        ],
        out_specs=[
            pl.BlockSpec((_P, C_OUT), lambda i: (i, 0)),
            pl.BlockSpec((pk, 4), lambda i: (i, 0)),
        ],
        out_shape=[
            jax.ShapeDtypeStruct((N, C_OUT), jnp.float32),
            jax.ShapeDtypeStruct((NK, 4), jnp.float32),
        ],
    )(q, pn16, pnr, kvr, w1, b1, w2, b2, wo, bo)


@jax.jit
def kernel(p, x, o, n, idx, Wq, Wk, Wv, W1, b1, W2, b2, Wo, bo):
    del o
    idx_flat = idx.astype(jnp.int32).reshape(NK)
    pn16 = jnp.concatenate(
        [p, n, jnp.zeros((N, 10), jnp.float32)], axis=1)
    pn6 = jnp.concatenate([p.T, n.T], axis=0).reshape(6 * N)
    wqkv = jnp.concatenate([Wq, Wk, Wv], axis=1)

    q, kv = _qkv_call(x, wqkv)
    kvr = _gather_kv_call(kv, idx_flat)
    pnr = _gather_geom_call(pn6, idx_flat).reshape(NK, 16)
    x_out, ppf_flat = _main_call(
        q, pn16, pnr, kvr, W1, b1.reshape(1, C_HID), W2,
        b2.reshape(1, C_HID), Wo, bo.reshape(1, C_OUT))
    return (x_out, idx, ppf_flat.reshape(N, K, 4))
